# Initial kernel scaffold; baseline (speedup 1.0000x reference)
import jax
import jax.numpy as jnp
from jax.experimental import pallas as pl


def kernel(node_features, adj, W1, b1, W2, b2):
    raise NotImplementedError("write your pallas kernel here")


# speedup vs baseline: 1.4098x; 1.4098x over previous
import functools

import jax
import jax.numpy as jnp
from jax.experimental import pallas as pl
from jax.experimental.pallas import tpu as pltpu

N = 10000
F_IN = 128
HID = 128
NCLASS = 16

ROWS = 400


def _proj_kernel(x_ref, w1_ref, y_ref):
    y_ref[:, :] = jnp.dot(x_ref[:, :], w1_ref[:, :],
                          preferred_element_type=jnp.float32)


def _hop1_kernel(adj_ref, y_ref, b1_ref, w2_ref, g_ref):
    a = adj_ref[:, :]
    s = jnp.sum(a, axis=1, keepdims=True) + 1e-8
    z = jnp.dot(a, y_ref[:, :], preferred_element_type=jnp.float32)
    h = jnp.maximum(z / s + b1_ref[:, :], 0.0)
    g_ref[:, :] = jnp.dot(h, w2_ref[:, :],
                          preferred_element_type=jnp.float32)


def _hop2_kernel(adj_ref, g_ref, b2_ref, o_ref):
    a = adj_ref[:, :]
    s = jnp.sum(a, axis=1, keepdims=True) + 1e-8
    z = jnp.dot(a, g_ref[:, :], preferred_element_type=jnp.float32)
    z = z / s + b2_ref[:, :]
    m = jnp.max(z, axis=1, keepdims=True)
    e = z - m
    o_ref[:, :] = e - jnp.log(jnp.sum(jnp.exp(e), axis=1, keepdims=True))


@functools.partial(jax.jit, static_argnames=("interpret",))
def _run(node_features, adj, W1, b1, W2, b2, interpret=False):
    b1r = b1.reshape(1, HID)
    b2r = b2.reshape(1, NCLASS)

    y = pl.pallas_call(
        _proj_kernel,
        out_shape=jax.ShapeDtypeStruct((N, HID), jnp.float32),
        interpret=interpret,
    )(node_features, W1)

    full = lambda *shape: pl.BlockSpec(shape, lambda i: (0,) * len(shape))
    rowtile = pl.BlockSpec((ROWS, N), lambda i: (i, 0))

    g = pl.pallas_call(
        _hop1_kernel,
        grid=(N // ROWS,),
        in_specs=[rowtile, full(N, HID), full(1, HID), full(HID, NCLASS)],
        out_specs=pl.BlockSpec((ROWS, NCLASS), lambda i: (i, 0)),
        out_shape=jax.ShapeDtypeStruct((N, NCLASS), jnp.float32),
        interpret=interpret,
    )(adj, y, b1r, W2)

    out = pl.pallas_call(
        _hop2_kernel,
        grid=(N // ROWS,),
        in_specs=[rowtile, full(N, NCLASS), full(1, NCLASS)],
        out_specs=pl.BlockSpec((ROWS, NCLASS), lambda i: (i, 0)),
        out_shape=jax.ShapeDtypeStruct((N, NCLASS), jnp.float32),
        interpret=interpret,
    )(adj, g, b2r)

    return out


def kernel(node_features, adj, W1, b1, W2, b2):
    return _run(node_features, adj, W1, b1, W2, b2)
